# Initial kernel scaffold; baseline (speedup 1.0000x reference)
import jax
import jax.numpy as jnp
from jax.experimental import pallas as pl


def kernel(x, emb_table, W, b):
    raise NotImplementedError("write your pallas kernel here")


# speedup vs baseline: 2.7631x; 2.7631x over previous
import functools

import jax
import jax.numpy as jnp
from jax import lax
from jax.experimental import pallas as pl
from jax.experimental.pallas import tpu as pltpu
from jax.experimental.pallas import tpu_sc as plsc

_NC = 2
_NS = 16
_NW = _NC * _NS
_LANES = 16


def _exp_table(emb_table, W, bias):
    V, D = emb_table.shape
    tile = 2000
    assert V % tile == 0

    def body(emb_ref, w_ref, b_ref, out_ref):
        y = lax.dot_general(
            emb_ref[...], w_ref[...],
            dimension_numbers=(((1,), (1,)), ((), ())),
            preferred_element_type=jnp.float32,
        )
        out_ref[...] = jnp.exp(y + b_ref[...])

    return pl.pallas_call(
        body,
        grid=(V // tile,),
        in_specs=[
            pl.BlockSpec((tile, D), lambda i: (i, 0)),
            pl.BlockSpec((D, D), lambda i: (0, 0)),
            pl.BlockSpec((1, D), lambda i: (0, 0)),
        ],
        out_specs=pl.BlockSpec((tile, D), lambda i: (i, 0)),
        out_shape=jax.ShapeDtypeStruct((V, D), jnp.float32),
    )(emb_table, W, bias.reshape(1, D))


def _sc_softmax_gather(E, xr, B, L, D, chunk):
    n_chunks = L // chunk
    bpw = B // _NW
    assert B % _NW == 0 and D % _LANES == 0
    ncol = D // _LANES
    mesh = plsc.VectorSubcoreMesh(core_axis_name="c", subcore_axis_name="s")

    @functools.partial(
        pl.kernel,
        mesh=mesh,
        out_type=jax.ShapeDtypeStruct((B * L, D), jnp.float32),
        scratch_types=[
            pltpu.VMEM((n_chunks, chunk), jnp.int32),
            pltpu.VMEM((L, D), jnp.float32),
            pltpu.SemaphoreType.DMA,
        ],
        compiler_params=pltpu.CompilerParams(use_tc_tiling_on_sc=False),
    )
    def body(e_hbm, xr_hbm, out_hbm, idx_v, rows_v, sem):
        wid = lax.axis_index("s") * _NC + lax.axis_index("c")

        def batch_body(i, carry):
            bb = wid * bpw + i
            pltpu.sync_copy(xr_hbm.at[pl.ds(bb * n_chunks, n_chunks)], idx_v)
            copies = [
                pltpu.async_copy(
                    e_hbm.at[idx_v.at[j]],
                    rows_v.at[pl.ds(j * chunk, chunk)],
                    sem,
                )
                for j in range(n_chunks)
            ]
            for cp in copies:
                cp.wait()

            zero = jnp.zeros((_LANES,), jnp.float32)

            def sum_body(r, accs):
                return tuple(
                    accs[c] + rows_v[r, pl.ds(c * _LANES, _LANES)]
                    for c in range(ncol)
                )

            sums = lax.fori_loop(0, L, sum_body, (zero,) * ncol)
            invs = tuple(1.0 / s for s in sums)

            def scale_body(r, carry2):
                for c in range(ncol):
                    sl = pl.ds(c * _LANES, _LANES)
                    rows_v[r, sl] = rows_v[r, sl] * invs[c]
                return carry2

            lax.fori_loop(0, L, scale_body, 0)
            pltpu.sync_copy(rows_v, out_hbm.at[pl.ds(bb * L, L)])
            return carry

        lax.fori_loop(0, bpw, batch_body, 0)

    return body(E, xr)


def kernel(x, emb_table, W, b):
    B, L = x.shape
    V, D = emb_table.shape
    chunk = next(c for c in range(min(L, 128), 0, -1) if L % c == 0)
    E = _exp_table(emb_table, W, b)
    xr = x.astype(jnp.int32).reshape(B * L // chunk, chunk)
    out_flat = _sc_softmax_gather(E, xr, B, L, D, chunk)
    return out_flat.reshape(B, L, D)


# speedup vs baseline: 3.7019x; 1.3397x over previous
import functools

import jax
import jax.numpy as jnp
from jax import lax
from jax.experimental import pallas as pl
from jax.experimental.pallas import tpu as pltpu
from jax.experimental.pallas import tpu_sc as plsc

_NC = 2
_NS = 16
_NW = _NC * _NS
_LANES = 16


def _exp_table(emb_table, W, bias):
    V, D = emb_table.shape
    tile = 2000
    assert V % tile == 0

    def body(emb_ref, w_ref, b_ref, out_ref):
        y = lax.dot_general(
            emb_ref[...], w_ref[...],
            dimension_numbers=(((1,), (1,)), ((), ())),
            preferred_element_type=jnp.float32,
        )
        out_ref[...] = jnp.exp(y + b_ref[...])

    return pl.pallas_call(
        body,
        grid=(V // tile,),
        in_specs=[
            pl.BlockSpec((tile, D), lambda i: (i, 0)),
            pl.BlockSpec((D, D), lambda i: (0, 0)),
            pl.BlockSpec((1, D), lambda i: (0, 0)),
        ],
        out_specs=pl.BlockSpec((tile, D), lambda i: (i, 0)),
        out_shape=jax.ShapeDtypeStruct((V, D), jnp.float32),
    )(emb_table, W, bias.reshape(1, D))


def _sc_softmax_gather(E, xr, B, L, D, chunk):
    n_chunks = L // chunk
    bpw = B // _NW
    assert B % _NW == 0 and D % _LANES == 0
    ncol = D // _LANES
    NB = 4
    assert bpw % NB == 0 and bpw // NB >= 2
    R = bpw // NB
    U = 8
    assert L % U == 0
    mesh = plsc.VectorSubcoreMesh(core_axis_name="c", subcore_axis_name="s")

    @functools.partial(
        pl.kernel,
        mesh=mesh,
        out_type=jax.ShapeDtypeStruct((B * L, D), jnp.float32),
        scratch_types=[
            pltpu.VMEM((NB, n_chunks, chunk), jnp.int32),
            pltpu.VMEM((NB, L, D), jnp.float32),
            pltpu.VMEM((NB, L, D), jnp.float32),
        ]
        + [pltpu.SemaphoreType.DMA] * (2 * NB),
        compiler_params=pltpu.CompilerParams(use_tc_tiling_on_sc=False),
    )
    def body(e_hbm, xr_hbm, out_hbm, idx_v, g_v, o_v, *sems):
        gsem = sems[:NB]
        osem = sems[NB:]
        wid = lax.axis_index("s") * _NC + lax.axis_index("c")
        base = wid * bpw

        def issue_gather(u, b):
            pltpu.sync_copy(
                xr_hbm.at[pl.ds((base + b) * n_chunks, n_chunks)],
                idx_v.at[u],
            )
            for j in range(n_chunks):
                pltpu.async_copy(
                    e_hbm.at[idx_v.at[u, j]],
                    g_v.at[u, pl.ds(j * chunk, chunk)],
                    gsem[u],
                )

        def wait_gather(u):
            for j in range(n_chunks):
                pltpu.make_async_copy(
                    e_hbm.at[idx_v.at[u, j]],
                    g_v.at[u, pl.ds(j * chunk, chunk)],
                    gsem[u],
                ).wait()

        def out_slice(b):
            return out_hbm.at[pl.ds((base + b) * L, L)]

        def compute(u):
            g = g_v.at[u]
            o = o_v.at[u]
            zero = jnp.zeros((_LANES,), jnp.float32)

            def sum_body(t, accs):
                accs = list(accs)
                for uu in range(U):
                    for c in range(ncol):
                        p = (uu % 2) * ncol + c
                        accs[p] = accs[p] + g[t * U + uu, pl.ds(c * _LANES, _LANES)]
                return tuple(accs)

            accs = lax.fori_loop(0, L // U, sum_body, (zero,) * (2 * ncol))
            invs = [1.0 / (accs[c] + accs[ncol + c]) for c in range(ncol)]

            def scale_body(t, carry):
                for uu in range(U):
                    for c in range(ncol):
                        sl = pl.ds(c * _LANES, _LANES)
                        o[t * U + uu, sl] = g[t * U + uu, sl] * invs[c]
                return carry

            lax.fori_loop(0, L // U, scale_body, 0)

        def process(u, b, first, issue_next):
            wait_gather(u)
            if not first:
                pltpu.make_async_copy(o_v.at[u], out_slice(b - NB), osem[u]).wait()
            compute(u)
            pltpu.async_copy(o_v.at[u], out_slice(b), osem[u])
            if issue_next:
                issue_gather(u, b + NB)

        for u in range(NB):
            issue_gather(u, u)
        for u in range(NB):
            process(u, u, first=True, issue_next=True)

        def round_body(r, carry):
            for u in range(NB):
                process(u, r * NB + u, first=False, issue_next=True)
            return carry

        lax.fori_loop(1, R - 1, round_body, 0)

        for u in range(NB):
            process(u, (R - 1) * NB + u, first=False, issue_next=False)
        for u in range(NB):
            pltpu.make_async_copy(
                o_v.at[u], out_slice((R - 1) * NB + u), osem[u]
            ).wait()

    return body(E, xr)


def kernel(x, emb_table, W, b):
    B, L = x.shape
    V, D = emb_table.shape
    chunk = next(c for c in range(min(L, 128), 0, -1) if L % c == 0)
    E = _exp_table(emb_table, W, b)
    xr = x.astype(jnp.int32).reshape(B * L // chunk, chunk)
    out_flat = _sc_softmax_gather(E, xr, B, L, D, chunk)
    return out_flat.reshape(B, L, D)
